# baseline (speedup 1.0000x reference)
import jax
import jax.numpy as jnp
import numpy as np
from jax import lax
from jax.experimental import pallas as pl
from jax.experimental.pallas import tpu as pltpu
from jax.experimental.pallas import tpu_sc as plsc

BN, BC = 64, 768
CELLS_IN = 16
NUM_WORKERS = 32
N_PER_W = BN // NUM_WORKERS
CB = 128
NCH = BC // CB
LANES = 16

_TAB = np.zeros((128, LANES), dtype=np.int32)
for _v in range(120):
    _TAB[_v, :] = (_v % 30) * 4 + _v // 30


def _unpool_body(x_hbm, idx_hbm, ptab_hbm, out_hbm,
                 xb0, xb1, ib0, ib1, ob0, ob1, tbd2,
                 sx0, sx1, si0, si1, so0, so1):
    wid = lax.axis_index("s") * 2 + lax.axis_index("c")
    pltpu.sync_copy(ptab_hbm, tbd2)

    lanes = lax.iota(jnp.int32, LANES)
    zeros = jnp.zeros((LANES,), jnp.float32)
    bufs = ((xb0, ib0, ob0, sx0, si0, so0, wid * N_PER_W),
            (xb1, ib1, ob1, sx1, si1, so1, wid * N_PER_W + 1))

    def issue_in(ch, b):
        xb, ib, _, sx, si, _, n = bufs[b]
        c0 = ch * CB
        pltpu.async_copy(x_hbm.at[n, :, :, :, pl.ds(c0, CB)], xb, sx)
        pltpu.async_copy(idx_hbm.at[n, :, :, :, pl.ds(c0, CB)], ib, si)

    for b in range(2):
        issue_in(0, b)

    @pl.loop(0, NCH)
    def _ch(ch):
        c0 = ch * CB
        for b in range(2):
            xb, ib, ob, sx, si, so, n = bufs[b]

            @pl.when(ch > 0)
            def _():
                pltpu.make_async_copy(ob, out_hbm.at[n, :, :, pl.ds(c0, CB)], so).wait()

            @pl.loop(0, 30)
            def _zero(r):
                for d in range(4):
                    for k in range(CB // LANES):
                        ob[r, d, pl.ds(k * LANES, LANES)] = zeros

            pltpu.make_async_copy(x_hbm.at[n, :, :, :, pl.ds(c0, CB)], xb, sx).wait()
            pltpu.make_async_copy(idx_hbm.at[n, :, :, :, pl.ds(c0, CB)], ib, si).wait()

            xb2 = xb.reshape(CELLS_IN, CB)
            ib2 = ib.reshape(CELLS_IN, CB)
            ob2 = ob.reshape(30 * 4, CB)

            for k in range(CB // LANES):
                cols = lanes + k * LANES
                for p in range(CELLS_IN):
                    vals = xb2[p, pl.ds(k * LANES, LANES)]
                    idxv = ib2[p, pl.ds(k * LANES, LANES)]
                    rowv = plsc.load_gather(tbd2, [idxv, lanes])
                    plsc.store_scatter(ob2, [rowv, cols], vals)

            @pl.when(ch < NCH - 1)
            def _():
                issue_in(ch + 1, b)

            pltpu.async_copy(ob, out_hbm.at[n, :, :, pl.ds(c0, CB)], so)

    for b in range(2):
        xb, ib, ob, sx, si, so, n = bufs[b]
        pltpu.make_async_copy(ob, out_hbm.at[n, :, :, pl.ds((NCH - 1) * CB, CB)], so).wait()


@jax.jit
def _unpool(x5, i5, ptab):
    mesh = plsc.VectorSubcoreMesh(core_axis_name="c", subcore_axis_name="s")
    return pl.kernel(
        _unpool_body,
        out_type=jax.ShapeDtypeStruct((BN, 30, 4, BC), jnp.float32),
        mesh=mesh,
        scratch_types=[
            pltpu.VMEM((2, 2, 4, CB), jnp.float32),
            pltpu.VMEM((2, 2, 4, CB), jnp.float32),
            pltpu.VMEM((2, 2, 4, CB), jnp.int32),
            pltpu.VMEM((2, 2, 4, CB), jnp.int32),
            pltpu.VMEM((30, 4, CB), jnp.float32),
            pltpu.VMEM((30, 4, CB), jnp.float32),
            pltpu.VMEM((128, LANES), jnp.int32),
            pltpu.SemaphoreType.DMA,
            pltpu.SemaphoreType.DMA,
            pltpu.SemaphoreType.DMA,
            pltpu.SemaphoreType.DMA,
            pltpu.SemaphoreType.DMA,
            pltpu.SemaphoreType.DMA,
        ],
        compiler_params=pltpu.CompilerParams(
            needs_layout_passes=False,
            disable_bounds_checks=True,
        ),
    )(x5, i5, ptab)


def kernel(x, indices):
    x5 = jnp.transpose(x, (0, 2, 3, 4, 1))
    i5 = jnp.transpose(indices.astype(jnp.int32), (0, 2, 3, 4, 1))
    out4 = _unpool(x5, i5, jnp.asarray(_TAB))
    out5 = out4.reshape(BN, 5, 6, 4, BC)
    return jnp.transpose(out5, (0, 4, 3, 1, 2))


# speedup vs baseline: 1.5958x; 1.5958x over previous
import jax
import jax.numpy as jnp
import numpy as np
from jax import lax
from jax.experimental import pallas as pl
from jax.experimental.pallas import tpu as pltpu
from jax.experimental.pallas import tpu_sc as plsc

BN, BC = 64, 768
CELLS_IN = 16
NUM_WORKERS = 32
N_PER_W = BN // NUM_WORKERS
CB = 128
NCH = BC // CB
LANES = 16

_TAB = np.zeros(128, dtype=np.int32)
for _v in range(120):
    _TAB[_v] = (_v % 30) * 4 + _v // 30


def _unpool_body(x_hbm, idx_hbm, ptab_hbm, out_hbm,
                 xb0, xb1, ib0, ib1, ob0, ob1, tbd,
                 sx0, sx1, si0, si1, so0, so1):
    wid = lax.axis_index("s") * 2 + lax.axis_index("c")
    pltpu.sync_copy(ptab_hbm, tbd)

    lanes = lax.iota(jnp.int32, LANES)
    zeros = jnp.zeros((LANES,), jnp.float32)
    bufs = ((xb0, ib0, ob0, sx0, si0, so0, wid * N_PER_W),
            (xb1, ib1, ob1, sx1, si1, so1, wid * N_PER_W + 1))

    def issue_in(ch, b):
        xb, ib, _, sx, si, _, n = bufs[b]
        c0 = ch * CB
        pltpu.async_copy(x_hbm.at[n, :, :, :, pl.ds(c0, CB)], xb, sx)
        pltpu.async_copy(idx_hbm.at[n, :, :, :, pl.ds(c0, CB)], ib, si)

    for b in range(2):
        issue_in(0, b)

    @pl.loop(0, NCH)
    def _ch(ch):
        c0 = ch * CB
        for b in range(2):
            xb, ib, ob, sx, si, so, n = bufs[b]

            @pl.when(ch > 0)
            def _():
                pltpu.make_async_copy(ob, out_hbm.at[n, :, :, pl.ds(c0, CB)], so).wait()

            @pl.loop(0, 30)
            def _zero(r):
                for d in range(4):
                    for k in range(CB // LANES):
                        ob[r, d, pl.ds(k * LANES, LANES)] = zeros

            pltpu.make_async_copy(x_hbm.at[n, :, :, :, pl.ds(c0, CB)], xb, sx).wait()
            pltpu.make_async_copy(idx_hbm.at[n, :, :, :, pl.ds(c0, CB)], ib, si).wait()

            xb2 = xb.reshape(CELLS_IN, CB)
            ib2 = ib.reshape(CELLS_IN, CB)
            ob2 = ob.reshape(30 * 4, CB)

            @plsc.parallel_loop(0, CB, step=LANES, unroll=4)
            def _sc(c):
                cols = lanes + c
                for p in range(CELLS_IN):
                    vals = xb2[p, pl.ds(c, LANES)]
                    idxv = ib2[p, pl.ds(c, LANES)]
                    rowv = plsc.load_gather(tbd, [idxv])
                    plsc.store_scatter(ob2, [rowv, cols], vals)

            @pl.when(ch < NCH - 1)
            def _():
                issue_in(ch + 1, b)

            pltpu.async_copy(ob, out_hbm.at[n, :, :, pl.ds(c0, CB)], so)

    for b in range(2):
        xb, ib, ob, sx, si, so, n = bufs[b]
        pltpu.make_async_copy(ob, out_hbm.at[n, :, :, pl.ds((NCH - 1) * CB, CB)], so).wait()


@jax.jit
def _unpool(x5, i5, ptab):
    mesh = plsc.VectorSubcoreMesh(core_axis_name="c", subcore_axis_name="s")
    return pl.kernel(
        _unpool_body,
        out_type=jax.ShapeDtypeStruct((BN, 30, 4, BC), jnp.float32),
        mesh=mesh,
        scratch_types=[
            pltpu.VMEM((2, 2, 4, CB), jnp.float32),
            pltpu.VMEM((2, 2, 4, CB), jnp.float32),
            pltpu.VMEM((2, 2, 4, CB), jnp.int32),
            pltpu.VMEM((2, 2, 4, CB), jnp.int32),
            pltpu.VMEM((30, 4, CB), jnp.float32),
            pltpu.VMEM((30, 4, CB), jnp.float32),
            pltpu.VMEM((128,), jnp.int32),
            pltpu.SemaphoreType.DMA,
            pltpu.SemaphoreType.DMA,
            pltpu.SemaphoreType.DMA,
            pltpu.SemaphoreType.DMA,
            pltpu.SemaphoreType.DMA,
            pltpu.SemaphoreType.DMA,
        ],
        compiler_params=pltpu.CompilerParams(
            needs_layout_passes=False,
            disable_bounds_checks=True,
        ),
    )(x5, i5, ptab)


def kernel(x, indices):
    x5 = jnp.transpose(x, (0, 2, 3, 4, 1))
    i5 = jnp.transpose(indices.astype(jnp.int32), (0, 2, 3, 4, 1))
    out4 = _unpool(x5, i5, jnp.asarray(_TAB))
    out5 = out4.reshape(BN, 5, 6, 4, BC)
    return jnp.transpose(out5, (0, 4, 3, 1, 2))


# speedup vs baseline: 1.5962x; 1.0003x over previous
import jax
import jax.numpy as jnp
import numpy as np
from jax import lax
from jax.experimental import pallas as pl
from jax.experimental.pallas import tpu as pltpu
from jax.experimental.pallas import tpu_sc as plsc

BN, BC = 64, 768
CELLS_IN = 16
NUM_WORKERS = 32
N_PER_W = BN // NUM_WORKERS
CB = 128
NCH = BC // CB
LANES = 16

_TAB = np.zeros(128, dtype=np.int32)
for _v in range(120):
    _TAB[_v] = (_v % 30) * 4 + _v // 30


def _unpool_body(x_hbm, idx_hbm, ptab_hbm, out_hbm,
                 xb0, xb1, ib0, ib1, ob0, ob1, tbd,
                 sx0, sx1, si0, si1, so0, so1):
    wid = lax.axis_index("s") * 2 + lax.axis_index("c")
    pltpu.sync_copy(ptab_hbm, tbd)

    lanes = lax.iota(jnp.int32, LANES)
    zeros = jnp.zeros((LANES,), jnp.float32)
    bufs = ((xb0, ib0, ob0, sx0, si0, so0, wid * N_PER_W),
            (xb1, ib1, ob1, sx1, si1, so1, wid * N_PER_W + 1))

    def issue_in(ch, b):
        xb, ib, _, sx, si, _, n = bufs[b]
        c0 = ch * CB
        pltpu.async_copy(x_hbm.at[n, :, :, :, pl.ds(c0, CB)], xb, sx)
        pltpu.async_copy(idx_hbm.at[n, :, :, :, pl.ds(c0, CB)], ib, si)

    for b in range(2):
        issue_in(0, b)

    @pl.loop(0, NCH)
    def _ch(ch):
        c0 = ch * CB
        for b in range(2):
            xb, ib, ob, sx, si, so, n = bufs[b]

            @pl.when(ch > 0)
            def _():
                pltpu.make_async_copy(ob, out_hbm.at[n, :, :, pl.ds(c0, CB)], so).wait()

            @plsc.parallel_loop(0, 30, step=1, unroll=2)
            def _zero(r):
                for d in range(4):
                    for k in range(CB // LANES):
                        ob[r, d, pl.ds(k * LANES, LANES)] = zeros

            pltpu.make_async_copy(x_hbm.at[n, :, :, :, pl.ds(c0, CB)], xb, sx).wait()
            pltpu.make_async_copy(idx_hbm.at[n, :, :, :, pl.ds(c0, CB)], ib, si).wait()

            xb2 = xb.reshape(CELLS_IN, CB)
            ib2 = ib.reshape(CELLS_IN, CB)
            ob2 = ob.reshape(30 * 4, CB)

            @plsc.parallel_loop(0, CB, step=LANES, unroll=4)
            def _sc(c):
                cols = lanes + c
                for p in range(CELLS_IN):
                    vals = xb2[p, pl.ds(c, LANES)]
                    idxv = ib2[p, pl.ds(c, LANES)]
                    rowv = plsc.load_gather(tbd, [idxv])
                    plsc.store_scatter(ob2, [rowv, cols], vals)

            @pl.when(ch < NCH - 1)
            def _():
                issue_in(ch + 1, b)

            pltpu.async_copy(ob, out_hbm.at[n, :, :, pl.ds(c0, CB)], so)

    for b in range(2):
        xb, ib, ob, sx, si, so, n = bufs[b]
        pltpu.make_async_copy(ob, out_hbm.at[n, :, :, pl.ds((NCH - 1) * CB, CB)], so).wait()


@jax.jit
def _unpool(x5, i5, ptab):
    mesh = plsc.VectorSubcoreMesh(core_axis_name="c", subcore_axis_name="s")
    return pl.kernel(
        _unpool_body,
        out_type=jax.ShapeDtypeStruct((BN, 30, 4, BC), jnp.float32),
        mesh=mesh,
        scratch_types=[
            pltpu.VMEM((2, 2, 4, CB), jnp.float32),
            pltpu.VMEM((2, 2, 4, CB), jnp.float32),
            pltpu.VMEM((2, 2, 4, CB), jnp.int32),
            pltpu.VMEM((2, 2, 4, CB), jnp.int32),
            pltpu.VMEM((30, 4, CB), jnp.float32),
            pltpu.VMEM((30, 4, CB), jnp.float32),
            pltpu.VMEM((128,), jnp.int32),
            pltpu.SemaphoreType.DMA,
            pltpu.SemaphoreType.DMA,
            pltpu.SemaphoreType.DMA,
            pltpu.SemaphoreType.DMA,
            pltpu.SemaphoreType.DMA,
            pltpu.SemaphoreType.DMA,
        ],
        compiler_params=pltpu.CompilerParams(
            needs_layout_passes=False,
            disable_bounds_checks=True,
        ),
    )(x5, i5, ptab)


def kernel(x, indices):
    x5 = jnp.transpose(x, (0, 2, 3, 4, 1))
    i5 = jnp.transpose(indices.astype(jnp.int32), (0, 2, 3, 4, 1))
    out4 = _unpool(x5, i5, jnp.asarray(_TAB))
    out5 = out4.reshape(BN, 5, 6, 4, BC)
    return jnp.transpose(out5, (0, 4, 3, 1, 2))
